# Initial kernel scaffold; baseline (speedup 1.0000x reference)
import jax
import jax.numpy as jnp
from jax.experimental import pallas as pl


def kernel(user_emb, video_emb, edge_index, edge_values, fc_w, fc_b, fcg_w, fcg_b):
    raise NotImplementedError("write your pallas kernel here")


# speedup vs baseline: 5.2255x; 5.2255x over previous
import functools

import jax
import jax.numpy as jnp
from jax import lax
from jax.experimental import pallas as pl
from jax.experimental.pallas import tpu as pltpu
from jax.experimental.pallas import tpu_sc as plsc

N_USERS = 6000
N_VIDEOS = 4000
N = N_USERS + N_VIDEOS
N_PAD = 10240
D = 128
G = 4
E = 320000

NC = 2
NS = 16
NW = NC * NS
CHUNK = 128
E_PAD = 327680
ROWS_PER_TILE = N_PAD // NS

_mesh = plsc.VectorSubcoreMesh(core_axis_name="c", subcore_axis_name="s")

_GDN = lax.GatherDimensionNumbers(
    offset_dims=(), collapsed_slice_dims=(0,), start_index_map=(0,))


def _lane_bcast(v, l):
    return lax.gather(v, jnp.full((16, 1), l, jnp.int32), _GDN, (1,),
                      mode=lax.GatherScatterMode.PROMISE_IN_BOUNDS)


def _scale_rows(rowsv, vbuf):
    def batch_body(b, carry):
        v = vbuf[pl.ds(b * 16, 16)]
        for l in range(16):
            sc = _lane_bcast(v, l)
            e = b * 16 + l
            for k in range(8):
                sl = pl.ds(k * 16, 16)
                rowsv[e, sl] = rowsv[e, sl] * sc
        return carry
    lax.fori_loop(0, CHUNK // 16, batch_body, 0)


def _zero_rowsv(rowsv):
    def zrow(r, carry):
        for k in range(8):
            rowsv[r, pl.ds(k * 16, 16)] = jnp.zeros((16,), jnp.float32)
        return carry
    lax.fori_loop(0, CHUNK, zrow, 0)


def _zero_acc_slice(rowsv, acc, sid):
    base = sid * ROWS_PER_TILE
    for t in range(ROWS_PER_TILE // CHUNK):
        pltpu.sync_copy(rowsv, acc.at[pl.ds(base + t * CHUNK, CHUNK)])


def _copy_out_slice(acc, out_slice_fn, sid):
    base = sid * ROWS_PER_TILE
    for t in range(ROWS_PER_TILE // CHUNK):
        sl = pl.ds(base + t * CHUNK, CHUNK)
        pltpu.sync_copy(acc.at[sl], out_slice_fn(sl))


@functools.partial(
    pl.kernel,
    mesh=_mesh,
    out_type=jax.ShapeDtypeStruct((NC, N_PAD, D), jnp.float32),
    scratch_types=[
        pltpu.VMEM((CHUNK,), jnp.int32),
        pltpu.VMEM((CHUNK,), jnp.int32),
        pltpu.VMEM((CHUNK,), jnp.float32),
        pltpu.VMEM((CHUNK, D), jnp.float32),
        pltpu.VMEM_SHARED((N_PAD, D), jnp.float32),
        pltpu.SemaphoreType.DMA,
    ],
)
def _sc_spmm0(col3, row3, vals3, x, out, cbuf, rbuf, vbuf, rowsv, acc, sem):
    cid = lax.axis_index("c")
    sid = lax.axis_index("s")
    w = cid * NS + sid
    n_chunks = E_PAD // NW // CHUNK

    _zero_rowsv(rowsv)
    _zero_acc_slice(rowsv, acc, sid)
    plsc.subcore_barrier()

    def chunk_body(j, carry):
        pltpu.sync_copy(col3.at[w, j], cbuf)
        pltpu.sync_copy(row3.at[w, j], rbuf)
        pltpu.sync_copy(vals3.at[w, j], vbuf)
        pltpu.async_copy(x.at[cbuf], rowsv, sem).wait()
        _scale_rows(rowsv, vbuf)
        pltpu.sync_copy(rowsv, acc.at[rbuf], add=True)
        return carry
    lax.fori_loop(0, n_chunks, chunk_body, 0)

    plsc.subcore_barrier()
    _copy_out_slice(acc, lambda sl: out.at[cid, sl], sid)


@functools.partial(
    pl.kernel,
    mesh=_mesh,
    out_type=jax.ShapeDtypeStruct((G, N_PAD, D), jnp.float32),
    scratch_types=[
        pltpu.VMEM((CHUNK,), jnp.int32),
        pltpu.VMEM((CHUNK,), jnp.int32),
        pltpu.VMEM((CHUNK,), jnp.float32),
        pltpu.VMEM((CHUNK, D), jnp.float32),
        pltpu.VMEM_SHARED((N_PAD, D), jnp.float32),
        pltpu.SemaphoreType.DMA,
    ],
)
def _sc_spmm_groups(col3, row3, vals3, xg, out, cbuf, rbuf, vbuf, rowsv, acc,
                    sem):
    cid = lax.axis_index("c")
    sid = lax.axis_index("s")
    n_chunks = E_PAD // NS // CHUNK

    _zero_rowsv(rowsv)

    for q in range(G // NC):
        g = cid * (G // NC) + q
        goff = jnp.full((16,), g * N_PAD, jnp.int32)
        _zero_acc_slice(rowsv, acc, sid)
        plsc.subcore_barrier()

        def chunk_body(j, carry):
            pltpu.sync_copy(col3.at[sid, j], cbuf)
            pltpu.sync_copy(row3.at[sid, j], rbuf)
            pltpu.sync_copy(vals3.at[sid, j], vbuf)
            for k in range(8):
                sl = pl.ds(k * 16, 16)
                cbuf[sl] = cbuf[sl] + goff
            pltpu.async_copy(xg.at[cbuf], rowsv, sem).wait()
            _scale_rows(rowsv, vbuf)
            pltpu.sync_copy(rowsv, acc.at[rbuf], add=True)
            return carry
        lax.fori_loop(0, n_chunks, chunk_body, 0)

        plsc.subcore_barrier()
        _copy_out_slice(acc, lambda sl: out.at[g, sl], sid)
        plsc.subcore_barrier()
        _zero_rowsv(rowsv)


_BLK = 2048
_GRID = N_PAD // _BLK


def _tc1_body(ego_ref, side_ref, fcw_ref, fcb_ref, fgw_ref, fgb_ref,
              x1_ref, oh_ref):
    i = pl.program_id(0)
    ego = ego_ref[...]
    side = side_ref[0] + side_ref[1]
    h = jnp.dot(ego + side, fcw_ref[...], preferred_element_type=jnp.float32)
    h = h + fcb_ref[...]
    t = jnp.where(h >= 0, h, 0.01 * h)
    s = jnp.dot(t, fgw_ref[...], preferred_element_type=jnp.float32)
    s = s + fgb_ref[...]
    m = jnp.max(s, axis=1, keepdims=True)
    oh = (s == m).astype(jnp.float32)
    rowid = i * _BLK + lax.broadcasted_iota(jnp.int32, (_BLK, 1), 0)
    oh = jnp.where(rowid >= N_USERS, 1.0, oh)
    oh_ref[...] = oh
    for g in range(G):
        x1_ref[g] = oh[:, g:g + 1] * ego


def _tc2_body(y1_ref, oh_ref, x2_ref, ls1_ref):
    oh = oh_ref[...]
    acc = jnp.zeros((_BLK, D), jnp.float32)
    for g in range(G):
        e1 = oh[:, g:g + 1] * y1_ref[g]
        x2_ref[g] = e1
        acc = acc + e1
    ls1_ref[...] = acc


def _tc3_body(y2_ref, oh_ref, ego_ref, ls1_ref, out_ref):
    oh = oh_ref[...]
    acc = 4.0 * ego_ref[...] + ls1_ref[...]
    for g in range(G):
        acc = acc + oh[:, g:g + 1] * y2_ref[g]
    out_ref[...] = 0.2 * acc


def kernel(user_emb, video_emb, edge_index, edge_values, fc_w, fc_b, fcg_w, fcg_b):
    ego = jnp.concatenate(
        [user_emb, video_emb, jnp.zeros((N_PAD - N, D), jnp.float32)], axis=0)
    row = edge_index[0].astype(jnp.int32)
    col = edge_index[1].astype(jnp.int32)
    vals = edge_values

    pad = E_PAD - E
    rowp = jnp.concatenate([row, jnp.zeros((pad,), jnp.int32)])
    colp = jnp.concatenate([col, jnp.zeros((pad,), jnp.int32)])
    valsp = jnp.concatenate([vals, jnp.zeros((pad,), jnp.float32)])

    row32 = rowp.reshape(NW, -1, CHUNK)
    col32 = colp.reshape(NW, -1, CHUNK)
    vals32 = valsp.reshape(NW, -1, CHUNK)
    row16 = rowp.reshape(NS, -1, CHUNK)
    col16 = colp.reshape(NS, -1, CHUNK)
    vals16 = valsp.reshape(NS, -1, CHUNK)

    side2 = _sc_spmm0(col32, row32, vals32, ego)

    x1, oh = pl.pallas_call(
        _tc1_body,
        grid=(_GRID,),
        in_specs=[
            pl.BlockSpec((_BLK, D), lambda i: (i, 0)),
            pl.BlockSpec((NC, _BLK, D), lambda i: (0, i, 0)),
            pl.BlockSpec((D, D), lambda i: (0, 0)),
            pl.BlockSpec((1, D), lambda i: (0, 0)),
            pl.BlockSpec((D, G), lambda i: (0, 0)),
            pl.BlockSpec((1, G), lambda i: (0, 0)),
        ],
        out_specs=[
            pl.BlockSpec((G, _BLK, D), lambda i: (0, i, 0)),
            pl.BlockSpec((_BLK, G), lambda i: (i, 0)),
        ],
        out_shape=[
            jax.ShapeDtypeStruct((G, N_PAD, D), jnp.float32),
            jax.ShapeDtypeStruct((N_PAD, G), jnp.float32),
        ],
    )(ego, side2, fc_w, fc_b.reshape(1, D), fcg_w, fcg_b.reshape(1, G))

    y1 = _sc_spmm_groups(col16, row16, vals16, x1.reshape(G * N_PAD, D))

    x2, ls1 = pl.pallas_call(
        _tc2_body,
        grid=(_GRID,),
        in_specs=[
            pl.BlockSpec((G, _BLK, D), lambda i: (0, i, 0)),
            pl.BlockSpec((_BLK, G), lambda i: (i, 0)),
        ],
        out_specs=[
            pl.BlockSpec((G, _BLK, D), lambda i: (0, i, 0)),
            pl.BlockSpec((_BLK, D), lambda i: (i, 0)),
        ],
        out_shape=[
            jax.ShapeDtypeStruct((G, N_PAD, D), jnp.float32),
            jax.ShapeDtypeStruct((N_PAD, D), jnp.float32),
        ],
    )(y1, oh)

    y2 = _sc_spmm_groups(col16, row16, vals16, x2.reshape(G * N_PAD, D))

    out = pl.pallas_call(
        _tc3_body,
        grid=(_GRID,),
        in_specs=[
            pl.BlockSpec((G, _BLK, D), lambda i: (0, i, 0)),
            pl.BlockSpec((_BLK, G), lambda i: (i, 0)),
            pl.BlockSpec((_BLK, D), lambda i: (i, 0)),
            pl.BlockSpec((_BLK, D), lambda i: (i, 0)),
        ],
        out_specs=pl.BlockSpec((_BLK, D), lambda i: (i, 0)),
        out_shape=jax.ShapeDtypeStruct((N_PAD, D), jnp.float32),
    )(y2, oh, ego, ls1)

    return out[:N_USERS], out[N_USERS:N]


# speedup vs baseline: 8.5938x; 1.6446x over previous
import functools

import jax
import jax.numpy as jnp
from jax import lax
from jax.experimental import pallas as pl
from jax.experimental.pallas import tpu as pltpu
from jax.experimental.pallas import tpu_sc as plsc

N_USERS = 6000
N_VIDEOS = 4000
N = N_USERS + N_VIDEOS
N_PAD = 10240
D = 128
G = 4
E = 320000

NC = 2
NS = 16
NW = NC * NS
CHUNK = 64
E_PAD = 327680
ROWS_PER_TILE = N_PAD // NS
PAD_ROW = N_PAD - 8

_mesh = plsc.VectorSubcoreMesh(core_axis_name="c", subcore_axis_name="s")

_GDN = lax.GatherDimensionNumbers(
    offset_dims=(), collapsed_slice_dims=(0,), start_index_map=(0,))


def _lane_bcast(v, l):
    return lax.gather(v, jnp.full((16, 1), l, jnp.int32), _GDN, (1,),
                      mode=lax.GatherScatterMode.PROMISE_IN_BOUNDS)


def _scale_rows(rowsv, vbuf):
    def batch_body(b, carry):
        v = vbuf[pl.ds(b * 16, 16)]
        for l in range(16):
            sc = _lane_bcast(v, l)
            e = b * 16 + l
            for k in range(8):
                sl = pl.ds(k * 16, 16)
                rowsv[e, sl] = rowsv[e, sl] * sc
        return carry
    lax.fori_loop(0, CHUNK // 16, batch_body, 0)


def _zero_rowsv(rowsv):
    def zrow(r, carry):
        for k in range(8):
            rowsv[r, pl.ds(k * 16, 16)] = jnp.zeros((16,), jnp.float32)
        return carry
    lax.fori_loop(0, CHUNK, zrow, 0)


def _offset_cols(pbuf, goff):
    if goff is not None:
        for k in range(CHUNK // 16):
            sl = pl.ds(k * 16, 16)
            pbuf[0, sl] = pbuf[0, sl] + goff


def _edge_pass(epack, ev, x, acc, p, vb, rv, dbuf, gsems, ssems, isems,
               vsems, w, n_chunks, goff):
    no = n_chunks // 8

    pltpu.async_copy(rv[3], acc.at[dbuf], ssems[3], add=True)
    for c in range(3):
        pltpu.sync_copy(epack.at[w, c], p[c])
        pltpu.sync_copy(ev.at[w, c], vb[c])
        _offset_cols(p[c], goff)
    pltpu.async_copy(epack.at[w, 3], p[3], isems[1])
    pltpu.async_copy(ev.at[w, 3], vb[3], vsems[1])
    pltpu.async_copy(epack.at[w, 4], p[4], isems[0])
    pltpu.async_copy(ev.at[w, 4], vb[4], vsems[0])
    for c in range(3):
        pltpu.async_copy(x.at[p[c].at[0]], rv[c], gsems[c])

    def octet(jo, carry):
        for r in range(8):
            j = jo * 8 + r
            b = r % 4
            pltpu.make_async_copy(x.at[p[r].at[0]], rv[b], gsems[b]).wait()

            def prep():
                b3 = (r + 3) % 4
                p3 = p[(r + 3) % 8]
                pltpu.make_async_copy(rv[b3], acc.at[dbuf], ssems[b3]).wait()
                pltpu.make_async_copy(
                    epack.at[w, 0], p3, isems[(r + 3) % 2]).wait()
                pltpu.make_async_copy(
                    ev.at[w, 0], vb[(r + 3) % 8], vsems[(r + 3) % 2]).wait()
                _offset_cols(p3, goff)
                pltpu.async_copy(x.at[p3.at[0]], rv[b3], gsems[b3])
            pl.when(j + 3 < n_chunks)(prep)

            def fetch_next():
                pltpu.async_copy(epack.at[w, j + 5], p[(r + 5) % 8],
                                 isems[(r + 5) % 2])
                pltpu.async_copy(ev.at[w, j + 5], vb[(r + 5) % 8],
                                 vsems[(r + 5) % 2])
            pl.when(j + 5 < n_chunks)(fetch_next)

            _scale_rows(rv[b], vb[r])
            pltpu.async_copy(rv[b], acc.at[p[r].at[1]], ssems[b], add=True)
        return carry
    lax.fori_loop(0, no, octet, 0)

    for b in range(4):
        pltpu.make_async_copy(rv[b], acc.at[dbuf], ssems[b]).wait()


def _zero_acc_slice(rowsv, acc, sid):
    base = sid * ROWS_PER_TILE
    for t in range(ROWS_PER_TILE // CHUNK):
        pltpu.sync_copy(rowsv, acc.at[pl.ds(base + t * CHUNK, CHUNK)])


def _fill_dbuf(dbuf):
    for k in range(CHUNK // 16):
        dbuf[pl.ds(k * 16, 16)] = jnp.full((16,), PAD_ROW, jnp.int32)


_SC_SCRATCH = (
    [pltpu.VMEM((2, CHUNK), jnp.int32) for _ in range(8)]
    + [pltpu.VMEM((CHUNK,), jnp.float32) for _ in range(8)]
    + [pltpu.VMEM((CHUNK, D), jnp.float32) for _ in range(4)]
    + [pltpu.VMEM((CHUNK,), jnp.int32)]
    + [pltpu.SemaphoreType.DMA for _ in range(12)]
)


@functools.partial(
    pl.kernel,
    mesh=_mesh,
    out_type=jax.ShapeDtypeStruct((NC, N_PAD, D), jnp.float32),
    scratch_types=_SC_SCRATCH + [
        pltpu.VMEM_SHARED((N_PAD, D), jnp.float32),
    ],
)
def _sc_spmm0(ep, ev, x, out, *scr):
    cid = lax.axis_index("c")
    sid = lax.axis_index("s")
    w = cid * NS + sid
    p, vb, rv = scr[0:8], scr[8:16], scr[16:20]
    dbuf = scr[20]
    gsems, ssems = scr[21:25], scr[25:29]
    isems, vsems = scr[29:31], scr[31:33]
    acc = scr[33]

    for b in range(4):
        _zero_rowsv(rv[b])
    _fill_dbuf(dbuf)
    _zero_acc_slice(rv[0], acc, sid)
    plsc.subcore_barrier()

    _edge_pass(ep, ev, x, acc, p, vb, rv, dbuf, gsems, ssems, isems, vsems,
               w, E_PAD // NW // CHUNK, None)

    plsc.subcore_barrier()
    base = sid * ROWS_PER_TILE
    for t in range(ROWS_PER_TILE // CHUNK):
        sl = pl.ds(base + t * CHUNK, CHUNK)
        pltpu.sync_copy(acc.at[sl], out.at[cid, sl])


@functools.partial(
    pl.kernel,
    mesh=_mesh,
    out_type=jax.ShapeDtypeStruct((G, N_PAD, D), jnp.float32),
    scratch_types=_SC_SCRATCH + [
        pltpu.VMEM_SHARED((N_PAD, D), jnp.float32),
    ],
)
def _sc_spmm_groups(ep, ev, xg, out, *scr):
    cid = lax.axis_index("c")
    sid = lax.axis_index("s")
    p, vb, rv = scr[0:8], scr[8:16], scr[16:20]
    dbuf = scr[20]
    gsems, ssems = scr[21:25], scr[25:29]
    isems, vsems = scr[29:31], scr[31:33]
    acc = scr[33]
    n_chunks = E_PAD // NS // CHUNK

    _fill_dbuf(dbuf)

    def group_body(q, carry):
        g = cid * (G // NC) + q
        goff = jnp.full((16,), g * N_PAD, jnp.int32)
        for b in range(4):
            _zero_rowsv(rv[b])
        _zero_acc_slice(rv[0], acc, sid)
        plsc.subcore_barrier()

        _edge_pass(ep, ev, xg, acc, p, vb, rv, dbuf, gsems, ssems, isems,
                   vsems, sid, n_chunks, goff)

        plsc.subcore_barrier()
        base = sid * ROWS_PER_TILE
        for t in range(ROWS_PER_TILE // CHUNK):
            sl = pl.ds(base + t * CHUNK, CHUNK)
            pltpu.sync_copy(acc.at[sl], out.at[g, sl])
        plsc.subcore_barrier()
        return carry
    lax.fori_loop(0, G // NC, group_body, 0)


_BLK = 2048
_GRID = N_PAD // _BLK


def _tc1_body(ego_ref, side_ref, fcw_ref, fcb_ref, fgw_ref, fgb_ref,
              x1_ref, oh_ref):
    i = pl.program_id(0)
    ego = ego_ref[...]
    side = side_ref[0] + side_ref[1]
    h = jnp.dot(ego + side, fcw_ref[...], preferred_element_type=jnp.float32)
    h = h + fcb_ref[...]
    t = jnp.where(h >= 0, h, 0.01 * h)
    s = jnp.dot(t, fgw_ref[...], preferred_element_type=jnp.float32)
    s = s + fgb_ref[...]
    m = jnp.max(s, axis=1, keepdims=True)
    oh = (s == m).astype(jnp.float32)
    rowid = i * _BLK + lax.broadcasted_iota(jnp.int32, (_BLK, 1), 0)
    oh = jnp.where(rowid >= N_USERS, 1.0, oh)
    oh_ref[...] = oh
    for g in range(G):
        x1_ref[g] = oh[:, g:g + 1] * ego


def _tc2_body(y1_ref, oh_ref, x2_ref, ls1_ref):
    oh = oh_ref[...]
    acc = jnp.zeros((_BLK, D), jnp.float32)
    for g in range(G):
        e1 = oh[:, g:g + 1] * y1_ref[g]
        x2_ref[g] = e1
        acc = acc + e1
    ls1_ref[...] = acc


def _tc3_body(y2_ref, oh_ref, ego_ref, ls1_ref, out_ref):
    oh = oh_ref[...]
    acc = 4.0 * ego_ref[...] + ls1_ref[...]
    for g in range(G):
        acc = acc + oh[:, g:g + 1] * y2_ref[g]
    out_ref[...] = 0.2 * acc


def kernel(user_emb, video_emb, edge_index, edge_values, fc_w, fc_b, fcg_w, fcg_b):
    ego = jnp.concatenate(
        [user_emb, video_emb, jnp.zeros((N_PAD - N, D), jnp.float32)], axis=0)
    row = edge_index[0].astype(jnp.int32)
    col = edge_index[1].astype(jnp.int32)
    vals = edge_values

    pad = E_PAD - E
    rowp = jnp.concatenate([row, jnp.zeros((pad,), jnp.int32)])
    colp = jnp.concatenate([col, jnp.zeros((pad,), jnp.int32)])
    valsp = jnp.concatenate([vals, jnp.zeros((pad,), jnp.float32)])

    ep32 = jnp.stack([colp.reshape(NW, -1, CHUNK),
                      rowp.reshape(NW, -1, CHUNK)], axis=2)
    ep16 = jnp.stack([colp.reshape(NS, -1, CHUNK),
                      rowp.reshape(NS, -1, CHUNK)], axis=2)
    ev32 = valsp.reshape(NW, -1, CHUNK)
    ev16 = valsp.reshape(NS, -1, CHUNK)

    side2 = _sc_spmm0(ep32, ev32, ego)

    x1, oh = pl.pallas_call(
        _tc1_body,
        grid=(_GRID,),
        in_specs=[
            pl.BlockSpec((_BLK, D), lambda i: (i, 0)),
            pl.BlockSpec((NC, _BLK, D), lambda i: (0, i, 0)),
            pl.BlockSpec((D, D), lambda i: (0, 0)),
            pl.BlockSpec((1, D), lambda i: (0, 0)),
            pl.BlockSpec((D, G), lambda i: (0, 0)),
            pl.BlockSpec((1, G), lambda i: (0, 0)),
        ],
        out_specs=[
            pl.BlockSpec((G, _BLK, D), lambda i: (0, i, 0)),
            pl.BlockSpec((_BLK, G), lambda i: (i, 0)),
        ],
        out_shape=[
            jax.ShapeDtypeStruct((G, N_PAD, D), jnp.float32),
            jax.ShapeDtypeStruct((N_PAD, G), jnp.float32),
        ],
    )(ego, side2, fc_w, fc_b.reshape(1, D), fcg_w, fcg_b.reshape(1, G))

    y1 = _sc_spmm_groups(ep16, ev16, x1.reshape(G * N_PAD, D))

    x2, ls1 = pl.pallas_call(
        _tc2_body,
        grid=(_GRID,),
        in_specs=[
            pl.BlockSpec((G, _BLK, D), lambda i: (0, i, 0)),
            pl.BlockSpec((_BLK, G), lambda i: (i, 0)),
        ],
        out_specs=[
            pl.BlockSpec((G, _BLK, D), lambda i: (0, i, 0)),
            pl.BlockSpec((_BLK, D), lambda i: (i, 0)),
        ],
        out_shape=[
            jax.ShapeDtypeStruct((G, N_PAD, D), jnp.float32),
            jax.ShapeDtypeStruct((N_PAD, D), jnp.float32),
        ],
    )(y1, oh)

    y2 = _sc_spmm_groups(ep16, ev16, x2.reshape(G * N_PAD, D))

    out = pl.pallas_call(
        _tc3_body,
        grid=(_GRID,),
        in_specs=[
            pl.BlockSpec((G, _BLK, D), lambda i: (0, i, 0)),
            pl.BlockSpec((_BLK, G), lambda i: (i, 0)),
            pl.BlockSpec((_BLK, D), lambda i: (i, 0)),
            pl.BlockSpec((_BLK, D), lambda i: (i, 0)),
        ],
        out_specs=pl.BlockSpec((_BLK, D), lambda i: (i, 0)),
        out_shape=jax.ShapeDtypeStruct((N_PAD, D), jnp.float32),
    )(y2, oh, ego, ls1)

    return out[:N_USERS], out[N_USERS:N]
